# baseline (speedup 1.0000x reference)
import functools

import jax
import jax.numpy as jnp
from jax import lax
from jax.experimental import pallas as pl
from jax.experimental.pallas import tpu as pltpu
from jax.experimental.pallas import tpu_sc as plsc

_NUM_CORES = 2
_NUM_SUBCORES = 16
_NW = _NUM_CORES * _NUM_SUBCORES
_L = 16
_G = 256
_CA = 512

_S0 = 16384
_S1 = 50
_D = 64
_V = 1000000


@functools.lru_cache(maxsize=None)
def _fused_kernel():
    n_chunks = (_S0 * _S1) // _NW // _G
    s0_blocks = _S0 // _G
    n_full_a = (_V - _V % _CA) // _CA
    a_per_w = n_full_a // _NW
    a_rem = n_full_a % _NW
    tail = _V % _CA
    mesh = plsc.VectorSubcoreMesh(core_axis_name="c", subcore_axis_name="s")

    @functools.partial(
        pl.kernel,
        out_type=jax.ShapeDtypeStruct((_S1 * _D, _S0), jnp.float32),
        mesh=mesh,
        scratch_types=[
            pltpu.HBM((_V // 2, 2 * _D), jnp.float32),
            pltpu.VMEM((_D, _CA), jnp.float32),
            [pltpu.VMEM((_G, 2 * _D), jnp.float32)] * 2,
            pltpu.VMEM((_D, _G), jnp.float32),
            pltpu.VMEM((tail // 2, 2 * _D), jnp.float32),
            [pltpu.VMEM((_G,), jnp.int32)] * 2,
            [pltpu.VMEM((_G,), jnp.int32)] * 2,
            pltpu.SemaphoreType.DMA,
            pltpu.SemaphoreType.DMA((2,)),
            pltpu.SemaphoreType.DMA((2,)),
            pltpu.SemaphoreType.DMA((2,)),
            pltpu.SemaphoreType.REGULAR,
        ],
        compiler_params=pltpu.CompilerParams(
            use_tc_tiling_on_sc=True, needs_layout_passes=False
        ),
    )
    def k(lines_hbm, paroff_hbm, wt_hbm, wtail_hbm, out_hbm, wcat_hbm, buf_a,
          buf_g, stg, buf_t, lines_v, paroff_v, asem, lsem, gsem, osem,
          bar_sem):
        cid = lax.axis_index("c")
        sid = lax.axis_index("s")
        wid = sid * _NUM_CORES + cid
        iota = lax.broadcasted_iota(jnp.int32, (_L,), 0)

        def do_block(j0):
            pltpu.async_copy(wt_hbm.at[:, pl.ds(j0, _CA)], buf_a, asem).wait()

            def lbody(l, carry):
                for p in range(2):
                    cols = jnp.full((_L,), 2 * l + p, dtype=jnp.int32)
                    for g in range(_D // _L):
                        rows = iota + (g * _L)
                        v = plsc.load_gather(buf_a, [rows, cols])
                        buf_g[0][l, pl.ds(p * _D + g * _L, _L)] = v
                return carry

            lax.fori_loop(0, _CA // 2, lbody, 0)
            pltpu.async_copy(
                buf_g[0],
                wcat_hbm.at[pl.ds(pl.multiple_of(j0 // 2, _CA // 2), _CA // 2)],
                asem,
            ).wait()

        def abody(i, carry):
            do_block(pl.multiple_of((wid + _NW * i) * _CA, _CA))
            return carry

        lax.fori_loop(0, a_per_w, abody, 0)

        @pl.when(wid < a_rem)
        def _():
            do_block(pl.multiple_of((wid + _NW * a_per_w) * _CA, _CA))

        if tail:
            @pl.when(wid == _NW - 1)
            def _():
                pltpu.async_copy(wtail_hbm, buf_t, asem).wait()
                pltpu.async_copy(
                    buf_t, wcat_hbm.at[pl.ds((_V - tail) // 2, tail // 2)],
                    asem,
                ).wait()

        plsc.subcore_barrier()

        @pl.when(sid == 0)
        def _():
            pl.semaphore_signal(bar_sem, 1, core_index=1 - cid)
            pl.semaphore_wait(bar_sem, 1)

        plsc.subcore_barrier()

        def chunk(i, p):
            lv, pv, gb, sg = lines_v[p], paroff_v[p], buf_g[p], stg
            t = wid * n_chunks + i
            s1 = t // s0_blocks
            s0 = pl.multiple_of((t % s0_blocks) * _G, _G)
            pltpu.async_copy(lines_hbm.at[t], lv, lsem.at[p])
            pltpu.async_copy(paroff_hbm.at[t], pv, lsem.at[p])
            pltpu.make_async_copy(lines_hbm.at[t], lv, lsem.at[p]).wait()
            pltpu.make_async_copy(paroff_hbm.at[t], pv, lsem.at[p]).wait()
            pltpu.async_copy(wcat_hbm.at[lv], gb, gsem.at[p]).wait()

            def cbody(g16, carry2):
                rows = iota + g16 * _L
                par = pv[pl.ds(g16 * _L, _L)]

                def dbody(d, carry3):
                    cols = par + d
                    v = plsc.load_gather(gb, [rows, cols])
                    sg[d, pl.ds(g16 * _L, _L)] = v
                    return carry3

                lax.fori_loop(0, _D, dbody, 0)
                return carry2

            lax.fori_loop(0, _G // _L, cbody, 0)

            pltpu.async_copy(
                sg,
                out_hbm.at[pl.ds(pl.multiple_of(s1 * _D, _D), _D),
                           pl.ds(s0, _G)],
                osem.at[p],
            ).wait()

        def chunk2(i2, carry):
            for p in range(2):
                chunk(i2 * 2 + p, p)
            return carry

        lax.fori_loop(0, n_chunks // 2, chunk2, 0)

    return k


def kernel(id_tensor, W):
    idt = id_tensor.T.astype(jnp.int32)
    n = _S0 * _S1
    lines = (idt >> 1).reshape(n // _G, _G)
    paroff = ((idt & 1) << 6).reshape(n // _G, _G)
    wtail = W[_V - _V % 128:]
    out_t = _fused_kernel()(lines, paroff, W.T, wtail.reshape(-1, 2 * _D))
    return out_t.reshape(_S1, _D, _S0).transpose(2, 0, 1)


# speedup vs baseline: 1.5835x; 1.5835x over previous
import functools

import jax
import jax.numpy as jnp
from jax import lax
from jax.experimental import pallas as pl
from jax.experimental.pallas import tpu as pltpu
from jax.experimental.pallas import tpu_sc as plsc

_NUM_CORES = 2
_NUM_SUBCORES = 16
_NW = _NUM_CORES * _NUM_SUBCORES
_L = 16
_G = 256
_CA = 512

_S0 = 16384
_S1 = 50
_D = 64
_V = 1000000


@functools.lru_cache(maxsize=None)
def _fused_kernel():
    n_chunks = (_S0 * _S1) // _NW // _G
    s0_blocks = _S0 // _G
    n_full_a = (_V - _V % _CA) // _CA
    a_per_w = n_full_a // _NW
    a_rem = n_full_a % _NW
    tail = _V % _CA
    mesh = plsc.VectorSubcoreMesh(core_axis_name="c", subcore_axis_name="s")

    @functools.partial(
        pl.kernel,
        out_type=jax.ShapeDtypeStruct((_S1 * _D, _S0), jnp.float32),
        mesh=mesh,
        scratch_types=[
            pltpu.HBM((_V // 2, 2 * _D), jnp.float32),
            pltpu.VMEM((_D, _CA), jnp.float32),
            [pltpu.VMEM((_G, 2 * _D), jnp.float32)] * 2,
            pltpu.VMEM((_D, _G), jnp.float32),
            pltpu.VMEM((tail // 2, 2 * _D), jnp.float32),
            [pltpu.VMEM((_G,), jnp.int32)] * 2,
            [pltpu.VMEM((_G,), jnp.int32)] * 2,
            pltpu.SemaphoreType.DMA,
            pltpu.SemaphoreType.DMA((2,)),
            pltpu.SemaphoreType.DMA((2,)),
            pltpu.SemaphoreType.DMA((2,)),
            pltpu.SemaphoreType.REGULAR,
        ],
        compiler_params=pltpu.CompilerParams(
            use_tc_tiling_on_sc=True, needs_layout_passes=False
        ),
    )
    def k(lines_hbm, paroff_hbm, wt_hbm, wtail_hbm, out_hbm, wcat_hbm, buf_a,
          buf_g, stg, buf_t, lines_v, paroff_v, asem, lsem, gsem, osem,
          bar_sem):
        cid = lax.axis_index("c")
        sid = lax.axis_index("s")
        wid = sid * _NUM_CORES + cid
        iota = lax.broadcasted_iota(jnp.int32, (_L,), 0)

        def do_block(j0):
            pltpu.async_copy(wt_hbm.at[:, pl.ds(j0, _CA)], buf_a, asem).wait()

            @plsc.parallel_loop(0, _CA // 2, unroll=4)
            def lbody(l):
                for p in range(2):
                    cols = jnp.full((_L,), 2 * l + p, dtype=jnp.int32)
                    for g in range(_D // _L):
                        rows = iota + (g * _L)
                        v = plsc.load_gather(buf_a, [rows, cols])
                        buf_g[0][l, pl.ds(p * _D + g * _L, _L)] = v
            pltpu.async_copy(
                buf_g[0],
                wcat_hbm.at[pl.ds(pl.multiple_of(j0 // 2, _CA // 2), _CA // 2)],
                asem,
            ).wait()

        def abody(i, carry):
            do_block(pl.multiple_of((wid + _NW * i) * _CA, _CA))
            return carry

        lax.fori_loop(0, a_per_w, abody, 0)

        @pl.when(wid < a_rem)
        def _():
            do_block(pl.multiple_of((wid + _NW * a_per_w) * _CA, _CA))

        if tail:
            @pl.when(wid == _NW - 1)
            def _():
                pltpu.async_copy(wtail_hbm, buf_t, asem).wait()
                pltpu.async_copy(
                    buf_t, wcat_hbm.at[pl.ds((_V - tail) // 2, tail // 2)],
                    asem,
                ).wait()

        plsc.subcore_barrier()

        @pl.when(sid == 0)
        def _():
            pl.semaphore_signal(bar_sem, 1, core_index=1 - cid)
            pl.semaphore_wait(bar_sem, 1)

        plsc.subcore_barrier()

        def chunk(i, p):
            lv, pv, gb, sg = lines_v[p], paroff_v[p], buf_g[p], stg
            t = wid * n_chunks + i
            s1 = t // s0_blocks
            s0 = pl.multiple_of((t % s0_blocks) * _G, _G)
            pltpu.async_copy(lines_hbm.at[t], lv, lsem.at[p])
            pltpu.async_copy(paroff_hbm.at[t], pv, lsem.at[p])
            pltpu.make_async_copy(lines_hbm.at[t], lv, lsem.at[p]).wait()
            pltpu.make_async_copy(paroff_hbm.at[t], pv, lsem.at[p]).wait()
            pltpu.async_copy(wcat_hbm.at[lv], gb, gsem.at[p]).wait()

            def cbody(g16, carry2):
                rows = iota + g16 * _L
                par = pv[pl.ds(g16 * _L, _L)]

                @plsc.parallel_loop(0, _D, unroll=8)
                def dbody(d):
                    cols = par + d
                    v = plsc.load_gather(gb, [rows, cols])
                    sg[d, pl.ds(g16 * _L, _L)] = v

                return carry2

            lax.fori_loop(0, _G // _L, cbody, 0)

            pltpu.async_copy(
                sg,
                out_hbm.at[pl.ds(pl.multiple_of(s1 * _D, _D), _D),
                           pl.ds(s0, _G)],
                osem.at[p],
            ).wait()

        def chunk2(i2, carry):
            for p in range(2):
                chunk(i2 * 2 + p, p)
            return carry

        lax.fori_loop(0, n_chunks // 2, chunk2, 0)

    return k


def kernel(id_tensor, W):
    idt = id_tensor.T.astype(jnp.int32)
    n = _S0 * _S1
    lines = (idt >> 1).reshape(n // _G, _G)
    paroff = ((idt & 1) << 6).reshape(n // _G, _G)
    wtail = W[_V - _V % 128:]
    out_t = _fused_kernel()(lines, paroff, W.T, wtail.reshape(-1, 2 * _D))
    return out_t.reshape(_S1, _D, _S0).transpose(2, 0, 1)


# speedup vs baseline: 1.8692x; 1.1804x over previous
import functools

import jax
import jax.numpy as jnp
from jax import lax
from jax.experimental import pallas as pl
from jax.experimental.pallas import tpu as pltpu
from jax.experimental.pallas import tpu_sc as plsc

_NUM_CORES = 2
_NUM_SUBCORES = 16
_NW = _NUM_CORES * _NUM_SUBCORES
_L = 16
_G = 256
_CA = 512

_S0 = 16384
_S1 = 50
_D = 64
_V = 1000000


@functools.lru_cache(maxsize=None)
def _fused_kernel():
    n_chunks = (_S0 * _S1) // _NW // _G
    s0_blocks = _S0 // _G
    n_full_a = (_V - _V % _CA) // _CA
    a_per_w = n_full_a // _NW
    a_rem = n_full_a % _NW
    tail = _V % _CA
    mesh = plsc.VectorSubcoreMesh(core_axis_name="c", subcore_axis_name="s")

    @functools.partial(
        pl.kernel,
        out_type=jax.ShapeDtypeStruct((_S1 * _D, _S0), jnp.float32),
        mesh=mesh,
        scratch_types=[
            pltpu.HBM((_V // 2, 2 * _D), jnp.float32),
            pltpu.VMEM((_D, _CA), jnp.float32),
            [pltpu.VMEM((_G, 2 * _D), jnp.float32)] * 2,
            pltpu.VMEM((_D, _G), jnp.float32),
            pltpu.VMEM((tail // 2, 2 * _D), jnp.float32),
            [pltpu.VMEM((_G,), jnp.int32)] * 2,
            [pltpu.VMEM((_G,), jnp.int32)] * 2,
            pltpu.SemaphoreType.DMA,
            pltpu.SemaphoreType.DMA((2,)),
            pltpu.SemaphoreType.DMA((2,)),
            pltpu.SemaphoreType.DMA((2,)),
            pltpu.SemaphoreType.REGULAR,
        ],
        compiler_params=pltpu.CompilerParams(
            use_tc_tiling_on_sc=True, needs_layout_passes=False
        ),
    )
    def k(lines_hbm, paroff_hbm, wt_hbm, wtail_hbm, out_hbm, wcat_hbm, buf_a,
          buf_g, stg, buf_t, lines_v, paroff_v, asem, lsem, gsem, osem,
          bar_sem):
        cid = lax.axis_index("c")
        sid = lax.axis_index("s")
        wid = sid * _NUM_CORES + cid
        iota = lax.broadcasted_iota(jnp.int32, (_L,), 0)

        def a_wcat_dst(j0):
            return wcat_hbm.at[
                pl.ds(pl.multiple_of(j0 // 2, _CA // 2), _CA // 2)]

        def do_block(j0, bg, wait_prev_out, j0_prev):
            pltpu.async_copy(wt_hbm.at[:, pl.ds(j0, _CA)], buf_a, asem).wait()

            @plsc.parallel_loop(0, _CA // 2, unroll=4)
            def lbody(l):
                for p in range(2):
                    cols = jnp.full((_L,), 2 * l + p, dtype=jnp.int32)
                    for g in range(_D // _L):
                        rows = iota + (g * _L)
                        v = plsc.load_gather(buf_a, [rows, cols])
                        bg[l, pl.ds(p * _D + g * _L, _L)] = v
            if wait_prev_out:
                pltpu.make_async_copy(
                    buf_g[1], a_wcat_dst(j0_prev), osem.at[1]).wait()
            pltpu.async_copy(bg, a_wcat_dst(j0), osem.at[0]).wait()

        def a_j0(i):
            return pl.multiple_of((wid + _NW * i) * _CA, _CA)

        def abody(i2, carry):
            i = i2 * 2
            j0, j1 = a_j0(i), a_j0(i + 1)
            pltpu.async_copy(wt_hbm.at[:, pl.ds(j0, _CA)], buf_a, asem).wait()

            @plsc.parallel_loop(0, _CA // 2, unroll=4)
            def lb0(l):
                for p in range(2):
                    cols = jnp.full((_L,), 2 * l + p, dtype=jnp.int32)
                    for g in range(_D // _L):
                        rows = iota + (g * _L)
                        v = plsc.load_gather(buf_a, [rows, cols])
                        buf_g[0][l, pl.ds(p * _D + g * _L, _L)] = v
            @pl.when(i2 > 0)
            def _():
                pltpu.make_async_copy(
                    buf_g[1], a_wcat_dst(a_j0(i - 1)), osem.at[1]).wait()
            pltpu.async_copy(buf_g[0], a_wcat_dst(j0), osem.at[0])

            pltpu.async_copy(wt_hbm.at[:, pl.ds(j1, _CA)], buf_a, asem).wait()

            @plsc.parallel_loop(0, _CA // 2, unroll=4)
            def lb1(l):
                for p in range(2):
                    cols = jnp.full((_L,), 2 * l + p, dtype=jnp.int32)
                    for g in range(_D // _L):
                        rows = iota + (g * _L)
                        v = plsc.load_gather(buf_a, [rows, cols])
                        buf_g[1][l, pl.ds(p * _D + g * _L, _L)] = v
            pltpu.make_async_copy(buf_g[0], a_wcat_dst(j0), osem.at[0]).wait()
            pltpu.async_copy(buf_g[1], a_wcat_dst(j1), osem.at[1])
            return carry

        lax.fori_loop(0, a_per_w // 2, abody, 0)

        pltpu.make_async_copy(
            buf_g[1], a_wcat_dst(a_j0(2 * (a_per_w // 2) - 1)),
            osem.at[1]).wait()

        if a_per_w % 2:
            do_block(a_j0(a_per_w - 1), buf_g[0], False, 0)

        @pl.when(wid < a_rem)
        def _():
            do_block(a_j0(a_per_w), buf_g[0], False, 0)

        if tail:
            @pl.when(wid == _NW - 1)
            def _():
                pltpu.async_copy(wtail_hbm, buf_t, asem).wait()
                pltpu.async_copy(
                    buf_t, wcat_hbm.at[pl.ds((_V - tail) // 2, tail // 2)],
                    asem,
                ).wait()

        plsc.subcore_barrier()

        @pl.when(sid == 0)
        def _():
            pl.semaphore_signal(bar_sem, 1, core_index=1 - cid)
            pl.semaphore_wait(bar_sem, 1)

        plsc.subcore_barrier()

        def load_lines(i, p):
            t = wid * n_chunks + i
            pltpu.async_copy(lines_hbm.at[t], lines_v[p], lsem.at[p])
            pltpu.async_copy(paroff_hbm.at[t], paroff_v[p], lsem.at[p])

        def wait_lines(i, p):
            t = wid * n_chunks + i
            pltpu.make_async_copy(lines_hbm.at[t], lines_v[p], lsem.at[p]).wait()
            pltpu.make_async_copy(paroff_hbm.at[t], paroff_v[p], lsem.at[p]).wait()

        def fire_gather(p):
            pltpu.async_copy(wcat_hbm.at[lines_v[p]], buf_g[p], gsem.at[p])

        def wait_gather(p):
            pltpu.make_async_copy(
                wcat_hbm.at[lines_v[p]], buf_g[p], gsem.at[p]).wait()

        def out_dst(i):
            t = wid * n_chunks + i
            s1 = t // s0_blocks
            s0 = pl.multiple_of((t % s0_blocks) * _G, _G)
            return out_hbm.at[pl.ds(pl.multiple_of(s1 * _D, _D), _D),
                              pl.ds(s0, _G)]

        def transpose_chunk(p):
            pv, gb = paroff_v[p], buf_g[p]

            def cbody(g16, carry2):
                rows = iota + g16 * _L
                par = pv[pl.ds(g16 * _L, _L)]

                @plsc.parallel_loop(0, _D, unroll=8)
                def dbody(d):
                    cols = par + d
                    v = plsc.load_gather(gb, [rows, cols])
                    stg[d, pl.ds(g16 * _L, _L)] = v

                return carry2

            lax.fori_loop(0, _G // _L, cbody, 0)

        def chunk(i, p):
            @pl.when(i + 1 < n_chunks)
            def _():
                wait_lines(i + 1, 1 - p)
                fire_gather(1 - p)
            wait_gather(p)
            @pl.when(i > 0)
            def _():
                pltpu.make_async_copy(stg, out_dst(i - 1), osem.at[0]).wait()
            transpose_chunk(p)
            pltpu.async_copy(stg, out_dst(i), osem.at[0])
            @pl.when(i + 2 < n_chunks)
            def _():
                load_lines(i + 2, p)

        load_lines(0, 0)
        load_lines(1, 1)
        wait_lines(0, 0)
        fire_gather(0)

        def chunk2(i2, carry):
            for p in range(2):
                chunk(i2 * 2 + p, p)
            return carry

        lax.fori_loop(0, n_chunks // 2, chunk2, 0)
        pltpu.make_async_copy(stg, out_dst(n_chunks - 1), osem.at[0]).wait()

    return k


def kernel(id_tensor, W):
    idt = id_tensor.T.astype(jnp.int32)
    n = _S0 * _S1
    lines = (idt >> 1).reshape(n // _G, _G)
    paroff = ((idt & 1) << 6).reshape(n // _G, _G)
    wtail = W[_V - _V % 128:]
    out_t = _fused_kernel()(lines, paroff, W.T, wtail.reshape(-1, 2 * _D))
    return out_t.reshape(_S1, _D, _S0).transpose(2, 0, 1)


# speedup vs baseline: 2.6270x; 1.4054x over previous
import functools

import jax
import jax.numpy as jnp
from jax import lax
from jax.experimental import pallas as pl
from jax.experimental.pallas import tpu as pltpu
from jax.experimental.pallas import tpu_sc as plsc

_NUM_CORES = 2
_NUM_SUBCORES = 16
_NW = _NUM_CORES * _NUM_SUBCORES
_C = 128
_K = 4


@functools.lru_cache(maxsize=None)
def _gather_kernel(B, D):
    n_per_w = B // _NW
    n_chunks = n_per_w // _C
    n_groups = n_chunks // _K
    mesh = plsc.VectorSubcoreMesh(core_axis_name="c", subcore_axis_name="s")

    @functools.partial(
        pl.kernel,
        out_type=jax.ShapeDtypeStruct((_NW, n_groups, _K * _C, D), jnp.float32),
        mesh=mesh,
        scratch_types=[
            pltpu.VMEM((n_groups, _K * _C), jnp.int32),
            pltpu.VMEM((2, _K * _C, D), jnp.float32),
            pltpu.SemaphoreType.DMA((2,)),
            pltpu.SemaphoreType.DMA((2,)),
        ],
        compiler_params=pltpu.CompilerParams(use_tc_tiling_on_sc=False),
    )
    def k(idx_hbm, table_hbm, out_hbm, idx_v, rows_v, gsem, osem):
        wid = lax.axis_index("s") * _NUM_CORES + lax.axis_index("c")
        pltpu.sync_copy(idx_hbm.at[wid], idx_v)

        def fire_gathers(g, p):
            pltpu.async_copy(
                table_hbm.at[idx_v.at[g]],
                rows_v.at[p],
                gsem.at[p],
            )

        def drain_gathers(g, p):
            pltpu.make_async_copy(
                table_hbm.at[idx_v.at[g]],
                rows_v.at[p],
                gsem.at[p],
            ).wait()

        def fire_out(g, p):
            pltpu.async_copy(rows_v.at[p], out_hbm.at[wid, g], osem.at[p])

        def wait_out(g, p):
            pltpu.make_async_copy(rows_v.at[p], out_hbm.at[wid, g], osem.at[p]).wait()

        fire_gathers(0, 0)
        fire_gathers(1, 1)
        drain_gathers(0, 0)
        fire_out(0, 0)

        def body(i, carry):
            for p2 in range(2):
                g = 1 + i * 2 + p2
                p = (1 + p2) % 2
                nxt = 1 - p
                wait_out(g - 1, nxt)
                fire_gathers(g + 1, nxt)
                drain_gathers(g, p)
                fire_out(g, p)
            return carry

        lax.fori_loop(0, (n_groups - 2) // 2, body, 0)

        g_last = n_groups - 1
        drain_gathers(g_last, 1)
        fire_out(g_last, 1)
        wait_out(g_last - 1, 0)
        wait_out(g_last, 1)

    return k


def kernel(id_tensor, W):
    S0, S1 = id_tensor.shape
    B = S0 * S1
    D = W.shape[1]
    idx = id_tensor.reshape(_NW, (B // _NW) // (_K * _C), _K * _C)
    out = _gather_kernel(B, D)(idx, W)
    return out.reshape(S0, S1, D)
